# baseline (speedup 1.0000x reference)
import functools
import math

import jax
import jax.numpy as jnp
from jax import lax
from jax.experimental import pallas as pl
from jax.experimental.pallas import tpu as pltpu
from jax.experimental.pallas import tpu_sc as plsc

DIM = 64
HID = 128
G = 128
BN = 4096


def _xpose_body(t_ref, o_ref):
    x = t_ref[...].astype(jnp.bfloat16)
    y = jnp.transpose(x, (1, 0))
    o_ref[...] = jnp.concatenate([y[:BN // 2], y[BN // 2:]], axis=1)


def _transpose_pack(tabT):
    d, n = tabT.shape
    nb = (n + BN - 1) // BN
    return pl.pallas_call(
        _xpose_body,
        grid=(nb,),
        in_specs=[pl.BlockSpec((64, BN), lambda i: (0, i))],
        out_specs=pl.BlockSpec((BN // 2, 2 * DIM), lambda i: (i, 0)),
        out_shape=jax.ShapeDtypeStruct((nb * (BN // 2), 2 * DIM),
                                       jnp.bfloat16),
        compiler_params=pltpu.CompilerParams(
            dimension_semantics=("parallel",),
        ),
    )(tabT)


def _permute_ids(ids, nw, steps, B, L):
    t = ids.T.astype(jnp.int32)
    t = t.reshape(L, 2, B // 2)
    a = jnp.transpose(t, (0, 2, 1))
    v = (a // BN) * BN + (a % (BN // 2)) * 2 + (a % BN) // (BN // 2)
    return v.reshape(nw, steps, G)


def _sc_gather(ids_s, ids_g, spack, gpack, nw, steps):
    n_rows = nw * steps * G
    sview = spack.reshape(spack.shape[0] * 2, DIM)
    gview = gpack.reshape(gpack.shape[0] * 2, DIM)
    mesh = plsc.VectorSubcoreMesh(core_axis_name="c", subcore_axis_name="s")
    nc = mesh.num_cores

    def body(sid_hbm, gid_hbm, stab_hbm, gtab_hbm, outs_hbm, outg_hbm,
             sidx, gidx, bufs, bufg, gsem):
        wid = lax.axis_index("s") * nc + lax.axis_index("c")
        pltpu.sync_copy(sid_hbm.at[wid], sidx)
        pltpu.sync_copy(gid_hbm.at[wid], gidx)
        row0 = wid * (steps * G)

        pltpu.async_copy(stab_hbm.at[sidx.at[0]], bufs.at[0], gsem)
        pltpu.async_copy(gtab_hbm.at[gidx.at[0]], bufg.at[0], gsem)

        def step(j, carry):
            slot = lax.rem(j, 2)
            nxt = lax.rem(j + 1, 2)
            pltpu.make_async_copy(stab_hbm.at[sidx.at[j]], bufs.at[slot],
                                  gsem).wait()
            pltpu.make_async_copy(gtab_hbm.at[gidx.at[j]], bufg.at[slot],
                                  gsem).wait()

            @pl.when(j + 1 < steps)
            def _():
                pltpu.async_copy(stab_hbm.at[sidx.at[j + 1]], bufs.at[nxt],
                                 gsem)
                pltpu.async_copy(gtab_hbm.at[gidx.at[j + 1]], bufg.at[nxt],
                                 gsem)

            base = row0 + j * G
            pltpu.sync_copy(bufs.at[slot], outs_hbm.at[pl.ds(base, G)])
            pltpu.sync_copy(bufg.at[slot], outg_hbm.at[pl.ds(base, G)])
            return carry

        lax.fori_loop(0, steps, step, 0)

    f = pl.kernel(
        body,
        out_type=(
            jax.ShapeDtypeStruct((n_rows, DIM), jnp.bfloat16),
            jax.ShapeDtypeStruct((n_rows, DIM), jnp.bfloat16),
        ),
        mesh=mesh,
        scratch_types=[
            pltpu.VMEM((steps, G), jnp.int32),
            pltpu.VMEM((steps, G), jnp.int32),
            pltpu.VMEM((2, G, DIM), jnp.bfloat16),
            pltpu.VMEM((2, G, DIM), jnp.bfloat16),
            pltpu.SemaphoreType.DMA,
        ],
        compiler_params=pltpu.CompilerParams(use_tc_tiling_on_sc=False),
    )
    return f(ids_s, ids_g, sview, gview)


def _ln_mlp_half_t(x, w1g, b1bt, w2, b2t):
    mu = jnp.mean(x, axis=-1, keepdims=True)
    xc = x - mu
    var = jnp.mean(xc * xc, axis=-1, keepdims=True)
    y = xc * lax.rsqrt(var + 1e-5)
    ht = lax.dot_general(w1g, y, (((0,), (1,)), ((), ())),
                         preferred_element_type=jnp.float32) + b1bt
    ht = 0.5 * ht * (1.0 + lax.erf(ht * (1.0 / math.sqrt(2.0))))
    return lax.dot_general(w2, ht, (((0,), (0,)), ((), ())),
                           preferred_element_type=jnp.float32) + b2t


def _mlp_body(es_ref, eg_ref, w1g_ref, b1bt_ref, w2_ref, b2t_ref, o_ref):
    x = es_ref[...].astype(jnp.float32) + eg_ref[...].astype(jnp.float32)
    z0t = _ln_mlp_half_t(x[:, :DIM], w1g_ref[...], b1bt_ref[...], w2_ref[...],
                         b2t_ref[...])
    z1t = _ln_mlp_half_t(x[:, DIM:], w1g_ref[...], b1bt_ref[...], w2_ref[...],
                         b2t_ref[...])
    r = x.shape[0]
    o_ref[0, :, 0:r] = z0t
    o_ref[0, :, r:2 * r] = z1t


def _tc_mlp(es, eg, gamma, beta, W1, b1, W2, b2, B, L):
    n_wide = es.shape[0] // 2
    wide_per_l = B // 2
    esw = es.reshape(n_wide, 2 * DIM)
    egw = eg.reshape(n_wide, 2 * DIM)
    w1g = gamma[:, None] * W1
    b1bt = (beta @ W1 + b1).reshape(HID, 1)
    b2t = b2.reshape(DIM, 1)
    full = lambda shape: pl.BlockSpec(shape, lambda i: (0,) * len(shape))
    out = pl.pallas_call(
        _mlp_body,
        grid=(L,),
        in_specs=[
            pl.BlockSpec((wide_per_l, 2 * DIM), lambda i: (i, 0)),
            pl.BlockSpec((wide_per_l, 2 * DIM), lambda i: (i, 0)),
            full((DIM, HID)),
            full((HID, 1)),
            full((HID, DIM)),
            full((DIM, 1)),
        ],
        out_specs=pl.BlockSpec((1, DIM, B), lambda i: (i, 0, 0)),
        out_shape=jax.ShapeDtypeStruct((L, DIM, B), jnp.float32),
        compiler_params=pltpu.CompilerParams(
            dimension_semantics=("parallel",),
        ),
    )(esw, egw, w1g, b1bt, W2, b2t)
    return jnp.transpose(out, (2, 0, 1))


def kernel(station_ids, geometry_ids, station_table, geometry_table, gamma,
           beta, W1, b1, W2, b2):
    B, L = station_ids.shape
    n = B * L
    nw = 32
    steps = n // (nw * G)
    assert steps * nw * G == n

    ids_s = _permute_ids(station_ids, nw, steps, B, L)
    ids_g = _permute_ids(geometry_ids, nw, steps, B, L)
    spack = _transpose_pack(station_table.T)
    gpack = _transpose_pack(geometry_table.T)
    es, eg = _sc_gather(ids_s, ids_g, spack, gpack, nw, steps)
    return _tc_mlp(es, eg, gamma, beta, W1, b1, W2, b2, B, L)


# speedup vs baseline: 1.5329x; 1.5329x over previous
import functools
import math

import jax
import jax.numpy as jnp
from jax import lax
from jax.experimental import pallas as pl
from jax.experimental.pallas import tpu as pltpu
from jax.experimental.pallas import tpu_sc as plsc

DIM = 64
HID = 128
G = 128


def _sc_gather(ids_s, ids_g, station_table, geometry_table, nw, steps):
    n_rows = nw * steps * G
    mesh = plsc.VectorSubcoreMesh(core_axis_name="c", subcore_axis_name="s")
    nc = mesh.num_cores

    def body(sid_hbm, gid_hbm, stab_hbm, gtab_hbm, outs_hbm, outg_hbm,
             sidx, gidx, bufs, bufg, gsem, wsem):
        wid = lax.axis_index("s") * nc + lax.axis_index("c")
        pltpu.sync_copy(sid_hbm.at[wid], sidx)
        pltpu.sync_copy(gid_hbm.at[wid], gidx)
        row0 = wid * (steps * G)

        pltpu.async_copy(stab_hbm.at[sidx.at[0]], bufs.at[0], gsem)
        pltpu.async_copy(gtab_hbm.at[gidx.at[0]], bufg.at[0], gsem)

        def step(j, carry):
            slot = lax.rem(j, 2)
            nxt = lax.rem(j + 1, 2)
            pltpu.make_async_copy(stab_hbm.at[sidx.at[j]], bufs.at[slot],
                                  gsem).wait()
            pltpu.make_async_copy(gtab_hbm.at[gidx.at[j]], bufg.at[slot],
                                  gsem).wait()

            @pl.when(j + 1 < steps)
            def _():
                pltpu.async_copy(stab_hbm.at[sidx.at[j + 1]], bufs.at[nxt],
                                 gsem)
                pltpu.async_copy(gtab_hbm.at[gidx.at[j + 1]], bufg.at[nxt],
                                 gsem)

            base = row0 + j * G
            pltpu.sync_copy(bufs.at[slot], outs_hbm.at[pl.ds(base, G)])
            pltpu.sync_copy(bufg.at[slot], outg_hbm.at[pl.ds(base, G)])
            return carry

        lax.fori_loop(0, steps, step, 0)

    f = pl.kernel(
        body,
        out_type=(
            jax.ShapeDtypeStruct((n_rows, DIM), jnp.float32),
            jax.ShapeDtypeStruct((n_rows, DIM), jnp.float32),
        ),
        mesh=mesh,
        scratch_types=[
            pltpu.VMEM((steps, G), jnp.int32),
            pltpu.VMEM((steps, G), jnp.int32),
            pltpu.VMEM((2, G, DIM), jnp.float32),
            pltpu.VMEM((2, G, DIM), jnp.float32),
            pltpu.SemaphoreType.DMA,
            pltpu.SemaphoreType.DMA,
        ],
        compiler_params=pltpu.CompilerParams(use_tc_tiling_on_sc=False),
    )
    return f(ids_s, ids_g, station_table, geometry_table)


def _ln_mlp_half_t(x, w1g, b1bt, w2, b2t):
    mu = jnp.mean(x, axis=-1, keepdims=True)
    xc = x - mu
    var = jnp.mean(xc * xc, axis=-1, keepdims=True)
    y = xc * lax.rsqrt(var + 1e-5)
    ht = lax.dot_general(w1g, y, (((0,), (1,)), ((), ())),
                         preferred_element_type=jnp.float32) + b1bt
    ht = 0.5 * ht * (1.0 + lax.erf(ht * (1.0 / math.sqrt(2.0))))
    return lax.dot_general(w2, ht, (((0,), (0,)), ((), ())),
                           preferred_element_type=jnp.float32) + b2t


def _mlp_body(es_ref, eg_ref, w1g_ref, b1bt_ref, w2_ref, b2t_ref, o_ref):
    x = es_ref[...] + eg_ref[...]
    z0t = _ln_mlp_half_t(x[:, :DIM], w1g_ref[...], b1bt_ref[...], w2_ref[...],
                         b2t_ref[...])
    z1t = _ln_mlp_half_t(x[:, DIM:], w1g_ref[...], b1bt_ref[...], w2_ref[...],
                         b2t_ref[...])
    r = x.shape[0]
    o_ref[0, :, 0:r] = z0t
    o_ref[0, :, r:2 * r] = z1t


def _tc_mlp(es, eg, gamma, beta, W1, b1, W2, b2, B, L):
    n_wide = es.shape[0] // 2
    wide_per_l = B // 2
    esw = es.reshape(n_wide, 2 * DIM)
    egw = eg.reshape(n_wide, 2 * DIM)
    w1g = gamma[:, None] * W1
    b1bt = (beta @ W1 + b1).reshape(HID, 1)
    b2t = b2.reshape(DIM, 1)
    grid = (L,)
    full = lambda shape: pl.BlockSpec(shape, lambda i: (0,) * len(shape))
    out = pl.pallas_call(
        _mlp_body,
        grid=grid,
        in_specs=[
            pl.BlockSpec((wide_per_l, 2 * DIM), lambda i: (i, 0)),
            pl.BlockSpec((wide_per_l, 2 * DIM), lambda i: (i, 0)),
            full((DIM, HID)),
            full((HID, 1)),
            full((HID, DIM)),
            full((DIM, 1)),
        ],
        out_specs=pl.BlockSpec((1, DIM, B), lambda i: (i, 0, 0)),
        out_shape=jax.ShapeDtypeStruct((L, DIM, B), jnp.float32),
        compiler_params=pltpu.CompilerParams(
            dimension_semantics=("parallel",),
        ),
    )(esw, egw, w1g, b1bt, W2, b2t)
    return jnp.transpose(out, (2, 0, 1))


def _permute_ids(ids, nw, steps, B, L):
    t = ids.T.astype(jnp.int32)
    t = t.reshape(L, 2, B // 2)
    t = jnp.transpose(t, (0, 2, 1))
    return t.reshape(nw, steps, G)


def kernel(station_ids, geometry_ids, station_table, geometry_table, gamma,
           beta, W1, b1, W2, b2):
    B, L = station_ids.shape
    n = B * L
    nw = 32
    steps = n // (nw * G)
    assert steps * nw * G == n

    ids_s = _permute_ids(station_ids, nw, steps, B, L)
    ids_g = _permute_ids(geometry_ids, nw, steps, B, L)
    es, eg = _sc_gather(ids_s, ids_g, station_table, geometry_table, nw, steps)
    return _tc_mlp(es, eg, gamma, beta, W1, b1, W2, b2, B, L)


# speedup vs baseline: 1.6714x; 1.0904x over previous
import functools
import math

import jax
import jax.numpy as jnp
from jax import lax
from jax.experimental import pallas as pl
from jax.experimental.pallas import tpu as pltpu
from jax.experimental.pallas import tpu_sc as plsc

DIM = 64
HID = 128
G = 128
PC = 4096


def _wide_pack(tabT):
    d, n = tabT.shape
    nb = (n + 2 * PC - 1) // (2 * PC)
    nbi = (n + PC - 1) // PC
    ident = jnp.eye(DIM, dtype=jnp.float32)

    def body(t0_ref, t1_ref, i_ref, o_ref):
        a = lax.dot_general(t0_ref[...], i_ref[...], (((0,), (0,)), ((), ())),
                            preferred_element_type=jnp.float32)
        b = lax.dot_general(t1_ref[...], i_ref[...], (((0,), (0,)), ((), ())),
                            preferred_element_type=jnp.float32)
        o_ref[:, :DIM] = a
        o_ref[:, DIM:] = b

    return pl.pallas_call(
        body,
        grid=(nb,),
        in_specs=[
            pl.BlockSpec((DIM, PC), lambda i: (0, jnp.minimum(2 * i, nbi - 1))),
            pl.BlockSpec((DIM, PC),
                         lambda i: (0, jnp.minimum(2 * i + 1, nbi - 1))),
            pl.BlockSpec((DIM, DIM), lambda i: (0, 0)),
        ],
        out_specs=pl.BlockSpec((PC, 2 * DIM), lambda i: (i, 0)),
        out_shape=jax.ShapeDtypeStruct((nb * PC, 2 * DIM), jnp.float32),
        compiler_params=pltpu.CompilerParams(
            dimension_semantics=("parallel",),
        ),
    )(tabT, tabT, ident)


def _packed_row(r):
    blk = r // (2 * PC)
    rem = r % (2 * PC)
    return 2 * (blk * PC + rem % PC) + rem // PC


def _sc_gather(ids_s, ids_g, station_table, geometry_table, nw, steps):
    n_rows = nw * steps * G
    mesh = plsc.VectorSubcoreMesh(core_axis_name="c", subcore_axis_name="s")
    nc = mesh.num_cores

    def body(sid_hbm, gid_hbm, stab_hbm, gtab_hbm, outs_hbm, outg_hbm,
             sidx, gidx, bufs, bufg, gsem, wsem):
        wid = lax.axis_index("s") * nc + lax.axis_index("c")
        pltpu.sync_copy(sid_hbm.at[wid], sidx)
        pltpu.sync_copy(gid_hbm.at[wid], gidx)
        row0 = wid * (steps * G)

        pltpu.async_copy(stab_hbm.at[sidx.at[0]], bufs.at[0], gsem)
        pltpu.async_copy(gtab_hbm.at[gidx.at[0]], bufg.at[0], gsem)

        def step(j, carry):
            slot = lax.rem(j, 2)
            nxt = lax.rem(j + 1, 2)
            pltpu.make_async_copy(stab_hbm.at[sidx.at[j]], bufs.at[slot],
                                  gsem).wait()
            pltpu.make_async_copy(gtab_hbm.at[gidx.at[j]], bufg.at[slot],
                                  gsem).wait()

            @pl.when(j + 1 < steps)
            def _():
                pltpu.async_copy(stab_hbm.at[sidx.at[j + 1]], bufs.at[nxt],
                                 gsem)
                pltpu.async_copy(gtab_hbm.at[gidx.at[j + 1]], bufg.at[nxt],
                                 gsem)

            base = row0 + j * G
            pltpu.sync_copy(bufs.at[slot], outs_hbm.at[pl.ds(base, G)])
            pltpu.sync_copy(bufg.at[slot], outg_hbm.at[pl.ds(base, G)])
            return carry

        lax.fori_loop(0, steps, step, 0)

    f = pl.kernel(
        body,
        out_type=(
            jax.ShapeDtypeStruct((n_rows, DIM), jnp.float32),
            jax.ShapeDtypeStruct((n_rows, DIM), jnp.float32),
        ),
        mesh=mesh,
        scratch_types=[
            pltpu.VMEM((steps, G), jnp.int32),
            pltpu.VMEM((steps, G), jnp.int32),
            pltpu.VMEM((2, G, DIM), jnp.float32),
            pltpu.VMEM((2, G, DIM), jnp.float32),
            pltpu.SemaphoreType.DMA,
            pltpu.SemaphoreType.DMA,
        ],
        compiler_params=pltpu.CompilerParams(use_tc_tiling_on_sc=False),
    )
    return f(ids_s, ids_g, station_table, geometry_table)


def _ln_mlp_half_t(x, w1g, b1bt, w2, b2t):
    mu = jnp.mean(x, axis=-1, keepdims=True)
    xc = x - mu
    var = jnp.mean(xc * xc, axis=-1, keepdims=True)
    y = xc * lax.rsqrt(var + 1e-5)
    ht = lax.dot_general(w1g, y, (((0,), (1,)), ((), ())),
                         preferred_element_type=jnp.float32) + b1bt
    ht = 0.5 * ht * (1.0 + lax.erf(ht * (1.0 / math.sqrt(2.0))))
    return lax.dot_general(w2, ht, (((0,), (0,)), ((), ())),
                           preferred_element_type=jnp.float32) + b2t


def _mlp_body(es_ref, eg_ref, w1g_ref, b1bt_ref, w2_ref, b2t_ref, o_ref):
    x = es_ref[...] + eg_ref[...]
    z0t = _ln_mlp_half_t(x[:, :DIM], w1g_ref[...], b1bt_ref[...], w2_ref[...],
                         b2t_ref[...])
    z1t = _ln_mlp_half_t(x[:, DIM:], w1g_ref[...], b1bt_ref[...], w2_ref[...],
                         b2t_ref[...])
    r = x.shape[0]
    o_ref[0, :, 0:r] = z0t
    o_ref[0, :, r:2 * r] = z1t


def _tc_mlp(es, eg, gamma, beta, W1, b1, W2, b2, B, L):
    n_wide = es.shape[0] // 2
    wide_per_l = B // 2
    esw = es.reshape(n_wide, 2 * DIM)
    egw = eg.reshape(n_wide, 2 * DIM)
    w1g = gamma[:, None] * W1
    b1bt = (beta @ W1 + b1).reshape(HID, 1)
    b2t = b2.reshape(DIM, 1)
    grid = (L,)
    full = lambda shape: pl.BlockSpec(shape, lambda i: (0,) * len(shape))
    out = pl.pallas_call(
        _mlp_body,
        grid=grid,
        in_specs=[
            pl.BlockSpec((wide_per_l, 2 * DIM), lambda i: (i, 0)),
            pl.BlockSpec((wide_per_l, 2 * DIM), lambda i: (i, 0)),
            full((DIM, HID)),
            full((HID, 1)),
            full((HID, DIM)),
            full((DIM, 1)),
        ],
        out_specs=pl.BlockSpec((1, DIM, B), lambda i: (i, 0, 0)),
        out_shape=jax.ShapeDtypeStruct((L, DIM, B), jnp.float32),
        compiler_params=pltpu.CompilerParams(
            dimension_semantics=("parallel",),
        ),
    )(esw, egw, w1g, b1bt, W2, b2t)
    return jnp.transpose(out, (2, 0, 1))


def _permute_ids(ids, nw, steps, B, L):
    t = ids.T.astype(jnp.int32)
    t = t.reshape(L, 2, B // 2)
    t = jnp.transpose(t, (0, 2, 1))
    return _packed_row(t).reshape(nw, steps, G)


def kernel(station_ids, geometry_ids, station_table, geometry_table, gamma,
           beta, W1, b1, W2, b2):
    B, L = station_ids.shape
    n = B * L
    nw = 32
    steps = n // (nw * G)
    assert steps * nw * G == n

    ids_s = _permute_ids(station_ids, nw, steps, B, L)
    ids_g = _permute_ids(geometry_ids, nw, steps, B, L)
    spack = _wide_pack(station_table.T)
    gpack = _wide_pack(geometry_table.T)
    sview = spack.reshape(spack.shape[0] * 2, DIM)
    gview = gpack.reshape(gpack.shape[0] * 2, DIM)
    es, eg = _sc_gather(ids_s, ids_g, sview, gview, nw, steps)
    return _tc_mlp(es, eg, gamma, beta, W1, b1, W2, b2, B, L)


# speedup vs baseline: 1.7380x; 1.0398x over previous
import functools
import math

import jax
import jax.numpy as jnp
from jax import lax
from jax.experimental import pallas as pl
from jax.experimental.pallas import tpu as pltpu
from jax.experimental.pallas import tpu_sc as plsc

DIM = 64
HID = 128
G = 128
PC = 8192


def _wide_pack(tabT):
    d, n = tabT.shape
    nb = (n + 2 * PC - 1) // (2 * PC)
    nbi = (n + PC - 1) // PC
    ident = jnp.eye(DIM, dtype=jnp.float32)

    def body(t0_ref, t1_ref, i_ref, o_ref):
        a = lax.dot_general(t0_ref[...], i_ref[...], (((0,), (0,)), ((), ())),
                            preferred_element_type=jnp.float32)
        b = lax.dot_general(t1_ref[...], i_ref[...], (((0,), (0,)), ((), ())),
                            preferred_element_type=jnp.float32)
        o_ref[:, :DIM] = a
        o_ref[:, DIM:] = b

    return pl.pallas_call(
        body,
        grid=(nb,),
        in_specs=[
            pl.BlockSpec((DIM, PC), lambda i: (0, jnp.minimum(2 * i, nbi - 1))),
            pl.BlockSpec((DIM, PC),
                         lambda i: (0, jnp.minimum(2 * i + 1, nbi - 1))),
            pl.BlockSpec((DIM, DIM), lambda i: (0, 0)),
        ],
        out_specs=pl.BlockSpec((PC, 2 * DIM), lambda i: (i, 0)),
        out_shape=jax.ShapeDtypeStruct((nb * PC, 2 * DIM), jnp.float32),
        compiler_params=pltpu.CompilerParams(
            dimension_semantics=("parallel",),
        ),
    )(tabT, tabT, ident)


def _packed_row(r):
    blk = r // (2 * PC)
    rem = r % (2 * PC)
    return 2 * (blk * PC + rem % PC) + rem // PC


def _sc_gather(ids_s, ids_g, station_table, geometry_table, nw, steps):
    n_rows = nw * steps * G
    mesh = plsc.VectorSubcoreMesh(core_axis_name="c", subcore_axis_name="s")
    nc = mesh.num_cores

    def body(sid_hbm, gid_hbm, stab_hbm, gtab_hbm, outs_hbm, outg_hbm,
             sidx, gidx, bufs, bufg, gsem, wsem):
        wid = lax.axis_index("s") * nc + lax.axis_index("c")
        pltpu.sync_copy(sid_hbm.at[wid], sidx)
        pltpu.sync_copy(gid_hbm.at[wid], gidx)
        row0 = wid * (steps * G)

        pltpu.async_copy(stab_hbm.at[sidx.at[0]], bufs.at[0], gsem)
        pltpu.async_copy(gtab_hbm.at[gidx.at[0]], bufg.at[0], gsem)

        def step(j, carry):
            slot = lax.rem(j, 2)
            nxt = lax.rem(j + 1, 2)
            pltpu.make_async_copy(stab_hbm.at[sidx.at[j]], bufs.at[slot],
                                  gsem).wait()
            pltpu.make_async_copy(gtab_hbm.at[gidx.at[j]], bufg.at[slot],
                                  gsem).wait()

            @pl.when(j + 1 < steps)
            def _():
                pltpu.async_copy(stab_hbm.at[sidx.at[j + 1]], bufs.at[nxt],
                                 gsem)
                pltpu.async_copy(gtab_hbm.at[gidx.at[j + 1]], bufg.at[nxt],
                                 gsem)

            base = row0 + j * G
            pltpu.sync_copy(bufs.at[slot], outs_hbm.at[pl.ds(base, G)])
            pltpu.sync_copy(bufg.at[slot], outg_hbm.at[pl.ds(base, G)])
            return carry

        lax.fori_loop(0, steps, step, 0)

    f = pl.kernel(
        body,
        out_type=(
            jax.ShapeDtypeStruct((n_rows, DIM), jnp.float32),
            jax.ShapeDtypeStruct((n_rows, DIM), jnp.float32),
        ),
        mesh=mesh,
        scratch_types=[
            pltpu.VMEM((steps, G), jnp.int32),
            pltpu.VMEM((steps, G), jnp.int32),
            pltpu.VMEM((2, G, DIM), jnp.float32),
            pltpu.VMEM((2, G, DIM), jnp.float32),
            pltpu.SemaphoreType.DMA,
            pltpu.SemaphoreType.DMA,
        ],
        compiler_params=pltpu.CompilerParams(use_tc_tiling_on_sc=False),
    )
    return f(ids_s, ids_g, station_table, geometry_table)


def _ln_mlp_half_t(x, w1g, b1bt, w2, b2t):
    mu = jnp.mean(x, axis=-1, keepdims=True)
    xc = x - mu
    var = jnp.mean(xc * xc, axis=-1, keepdims=True)
    y = xc * lax.rsqrt(var + 1e-5)
    ht = lax.dot_general(w1g, y, (((0,), (1,)), ((), ())),
                         preferred_element_type=jnp.float32) + b1bt
    ht = 0.5 * ht * (1.0 + lax.erf(ht * (1.0 / math.sqrt(2.0))))
    return lax.dot_general(w2, ht, (((0,), (0,)), ((), ())),
                           preferred_element_type=jnp.float32) + b2t


def _mlp_body(es_ref, eg_ref, w1g_ref, b1bt_ref, w2_ref, b2t_ref, o_ref):
    x = es_ref[...] + eg_ref[...]
    z0t = _ln_mlp_half_t(x[:, :DIM], w1g_ref[...], b1bt_ref[...], w2_ref[...],
                         b2t_ref[...])
    z1t = _ln_mlp_half_t(x[:, DIM:], w1g_ref[...], b1bt_ref[...], w2_ref[...],
                         b2t_ref[...])
    r = x.shape[0]
    o_ref[0, :, 0:r] = z0t
    o_ref[0, :, r:2 * r] = z1t


def _tc_mlp(es, eg, gamma, beta, W1, b1, W2, b2, B, L):
    n_wide = es.shape[0] // 2
    wide_per_l = B // 2
    esw = es.reshape(n_wide, 2 * DIM)
    egw = eg.reshape(n_wide, 2 * DIM)
    w1g = gamma[:, None] * W1
    b1bt = (beta @ W1 + b1).reshape(HID, 1)
    b2t = b2.reshape(DIM, 1)
    grid = (L,)
    full = lambda shape: pl.BlockSpec(shape, lambda i: (0,) * len(shape))
    out = pl.pallas_call(
        _mlp_body,
        grid=grid,
        in_specs=[
            pl.BlockSpec((wide_per_l, 2 * DIM), lambda i: (i, 0)),
            pl.BlockSpec((wide_per_l, 2 * DIM), lambda i: (i, 0)),
            full((DIM, HID)),
            full((HID, 1)),
            full((HID, DIM)),
            full((DIM, 1)),
        ],
        out_specs=pl.BlockSpec((1, DIM, B), lambda i: (i, 0, 0)),
        out_shape=jax.ShapeDtypeStruct((L, DIM, B), jnp.float32),
        compiler_params=pltpu.CompilerParams(
            dimension_semantics=("parallel",),
        ),
    )(esw, egw, w1g, b1bt, W2, b2t)
    return jnp.transpose(out, (2, 0, 1))


def _permute_ids(ids, nw, steps, B, L):
    t = ids.T.astype(jnp.int32)
    t = t.reshape(L, 2, B // 2)
    t = jnp.transpose(t, (0, 2, 1))
    return _packed_row(t).reshape(nw, steps, G)


def kernel(station_ids, geometry_ids, station_table, geometry_table, gamma,
           beta, W1, b1, W2, b2):
    B, L = station_ids.shape
    n = B * L
    nw = 32
    steps = n // (nw * G)
    assert steps * nw * G == n

    ids_s = _permute_ids(station_ids, nw, steps, B, L)
    ids_g = _permute_ids(geometry_ids, nw, steps, B, L)
    spack = _wide_pack(station_table.T)
    gpack = _wide_pack(geometry_table.T)
    sview = spack.reshape(spack.shape[0] * 2, DIM)
    gview = gpack.reshape(gpack.shape[0] * 2, DIM)
    es, eg = _sc_gather(ids_s, ids_g, sview, gview, nw, steps)
    return _tc_mlp(es, eg, gamma, beta, W1, b1, W2, b2, B, L)


# speedup vs baseline: 1.7605x; 1.0129x over previous
import functools
import math

import jax
import jax.numpy as jnp
from jax import lax
from jax.experimental import pallas as pl
from jax.experimental.pallas import tpu as pltpu
from jax.experimental.pallas import tpu_sc as plsc

DIM = 64
HID = 128
G = 128
PC = 16384


def _wide_pack(tabT):
    d, n = tabT.shape
    nb = (n + 2 * PC - 1) // (2 * PC)
    nbi = (n + PC - 1) // PC
    ident = jnp.eye(DIM, dtype=jnp.float32)

    def body(t0_ref, t1_ref, i_ref, o_ref):
        a = lax.dot_general(t0_ref[...], i_ref[...], (((0,), (0,)), ((), ())),
                            preferred_element_type=jnp.float32)
        b = lax.dot_general(t1_ref[...], i_ref[...], (((0,), (0,)), ((), ())),
                            preferred_element_type=jnp.float32)
        o_ref[:, :DIM] = a
        o_ref[:, DIM:] = b

    return pl.pallas_call(
        body,
        grid=(nb,),
        in_specs=[
            pl.BlockSpec((DIM, PC), lambda i: (0, jnp.minimum(2 * i, nbi - 1))),
            pl.BlockSpec((DIM, PC),
                         lambda i: (0, jnp.minimum(2 * i + 1, nbi - 1))),
            pl.BlockSpec((DIM, DIM), lambda i: (0, 0)),
        ],
        out_specs=pl.BlockSpec((PC, 2 * DIM), lambda i: (i, 0)),
        out_shape=jax.ShapeDtypeStruct((nb * PC, 2 * DIM), jnp.float32),
        compiler_params=pltpu.CompilerParams(
            dimension_semantics=("parallel",),
        ),
    )(tabT, tabT, ident)


def _packed_row(r):
    blk = r // (2 * PC)
    rem = r % (2 * PC)
    return 2 * (blk * PC + rem % PC) + rem // PC


def _sc_gather(ids_s, ids_g, station_table, geometry_table, nw, steps):
    n_rows = nw * steps * G
    mesh = plsc.VectorSubcoreMesh(core_axis_name="c", subcore_axis_name="s")
    nc = mesh.num_cores

    def body(sid_hbm, gid_hbm, stab_hbm, gtab_hbm, outs_hbm, outg_hbm,
             sidx, gidx, bufs, bufg, gsem, wsem):
        wid = lax.axis_index("s") * nc + lax.axis_index("c")
        pltpu.sync_copy(sid_hbm.at[wid], sidx)
        pltpu.sync_copy(gid_hbm.at[wid], gidx)
        row0 = wid * (steps * G)

        pltpu.async_copy(stab_hbm.at[sidx.at[0]], bufs.at[0], gsem)
        pltpu.async_copy(gtab_hbm.at[gidx.at[0]], bufg.at[0], gsem)

        def step(j, carry):
            slot = lax.rem(j, 2)
            nxt = lax.rem(j + 1, 2)
            pltpu.make_async_copy(stab_hbm.at[sidx.at[j]], bufs.at[slot],
                                  gsem).wait()
            pltpu.make_async_copy(gtab_hbm.at[gidx.at[j]], bufg.at[slot],
                                  gsem).wait()

            @pl.when(j + 1 < steps)
            def _():
                pltpu.async_copy(stab_hbm.at[sidx.at[j + 1]], bufs.at[nxt],
                                 gsem)
                pltpu.async_copy(gtab_hbm.at[gidx.at[j + 1]], bufg.at[nxt],
                                 gsem)

            base = row0 + j * G
            pltpu.sync_copy(bufs.at[slot], outs_hbm.at[pl.ds(base, G)])
            pltpu.sync_copy(bufg.at[slot], outg_hbm.at[pl.ds(base, G)])
            return carry

        lax.fori_loop(0, steps, step, 0)

    f = pl.kernel(
        body,
        out_type=(
            jax.ShapeDtypeStruct((n_rows, DIM), jnp.float32),
            jax.ShapeDtypeStruct((n_rows, DIM), jnp.float32),
        ),
        mesh=mesh,
        scratch_types=[
            pltpu.VMEM((steps, G), jnp.int32),
            pltpu.VMEM((steps, G), jnp.int32),
            pltpu.VMEM((2, G, DIM), jnp.float32),
            pltpu.VMEM((2, G, DIM), jnp.float32),
            pltpu.SemaphoreType.DMA,
            pltpu.SemaphoreType.DMA,
        ],
        compiler_params=pltpu.CompilerParams(use_tc_tiling_on_sc=False),
    )
    return f(ids_s, ids_g, station_table, geometry_table)


def _ln_mlp_half_t(x, w1g, b1bt, w2, b2t):
    mu = jnp.mean(x, axis=-1, keepdims=True)
    xc = x - mu
    var = jnp.mean(xc * xc, axis=-1, keepdims=True)
    y = xc * lax.rsqrt(var + 1e-5)
    ht = lax.dot_general(w1g, y, (((0,), (1,)), ((), ())),
                         preferred_element_type=jnp.float32) + b1bt
    ht = 0.5 * ht * (1.0 + lax.erf(ht * (1.0 / math.sqrt(2.0))))
    return lax.dot_general(w2, ht, (((0,), (0,)), ((), ())),
                           preferred_element_type=jnp.float32) + b2t


def _mlp_body(es_ref, eg_ref, w1g_ref, b1bt_ref, w2_ref, b2t_ref, o_ref):
    x = es_ref[...] + eg_ref[...]
    z0t = _ln_mlp_half_t(x[:, :DIM], w1g_ref[...], b1bt_ref[...], w2_ref[...],
                         b2t_ref[...])
    z1t = _ln_mlp_half_t(x[:, DIM:], w1g_ref[...], b1bt_ref[...], w2_ref[...],
                         b2t_ref[...])
    r = x.shape[0]
    o_ref[0, :, 0:r] = z0t
    o_ref[0, :, r:2 * r] = z1t


def _tc_mlp(es, eg, gamma, beta, W1, b1, W2, b2, B, L):
    n_wide = es.shape[0] // 2
    wide_per_l = B // 2
    esw = es.reshape(n_wide, 2 * DIM)
    egw = eg.reshape(n_wide, 2 * DIM)
    w1g = gamma[:, None] * W1
    b1bt = (beta @ W1 + b1).reshape(HID, 1)
    b2t = b2.reshape(DIM, 1)
    grid = (L,)
    full = lambda shape: pl.BlockSpec(shape, lambda i: (0,) * len(shape))
    out = pl.pallas_call(
        _mlp_body,
        grid=grid,
        in_specs=[
            pl.BlockSpec((wide_per_l, 2 * DIM), lambda i: (i, 0)),
            pl.BlockSpec((wide_per_l, 2 * DIM), lambda i: (i, 0)),
            full((DIM, HID)),
            full((HID, 1)),
            full((HID, DIM)),
            full((DIM, 1)),
        ],
        out_specs=pl.BlockSpec((1, DIM, B), lambda i: (i, 0, 0)),
        out_shape=jax.ShapeDtypeStruct((L, DIM, B), jnp.float32),
        compiler_params=pltpu.CompilerParams(
            dimension_semantics=("parallel",),
        ),
    )(esw, egw, w1g, b1bt, W2, b2t)
    return jnp.transpose(out, (2, 0, 1))


def _permute_ids(ids, nw, steps, B, L):
    t = ids.T.astype(jnp.int32)
    t = t.reshape(L, 2, B // 2)
    t = jnp.transpose(t, (0, 2, 1))
    return _packed_row(t).reshape(nw, steps, G)


def kernel(station_ids, geometry_ids, station_table, geometry_table, gamma,
           beta, W1, b1, W2, b2):
    B, L = station_ids.shape
    n = B * L
    nw = 32
    steps = n // (nw * G)
    assert steps * nw * G == n

    ids_s = _permute_ids(station_ids, nw, steps, B, L)
    ids_g = _permute_ids(geometry_ids, nw, steps, B, L)
    spack = _wide_pack(station_table.T)
    gpack = _wide_pack(geometry_table.T)
    sview = spack.reshape(spack.shape[0] * 2, DIM)
    gview = gpack.reshape(gpack.shape[0] * 2, DIM)
    es, eg = _sc_gather(ids_s, ids_g, sview, gview, nw, steps)
    return _tc_mlp(es, eg, gamma, beta, W1, b1, W2, b2, B, L)


# speedup vs baseline: 1.7714x; 1.0062x over previous
import functools
import math

import jax
import jax.numpy as jnp
from jax import lax
from jax.experimental import pallas as pl
from jax.experimental.pallas import tpu as pltpu
from jax.experimental.pallas import tpu_sc as plsc

DIM = 64
HID = 128
G = 128
PC = 16384


def _wide_pack(tabT):
    d, n = tabT.shape
    nb = (n + 2 * PC - 1) // (2 * PC)
    nbi = (n + PC - 1) // PC
    ident = jnp.eye(DIM, dtype=jnp.float32)

    def body(t0_ref, t1_ref, i_ref, o_ref):
        a = lax.dot_general(t0_ref[...], i_ref[...], (((0,), (0,)), ((), ())),
                            preferred_element_type=jnp.float32)
        b = lax.dot_general(t1_ref[...], i_ref[...], (((0,), (0,)), ((), ())),
                            preferred_element_type=jnp.float32)
        o_ref[:, :DIM] = a
        o_ref[:, DIM:] = b

    return pl.pallas_call(
        body,
        grid=(nb,),
        in_specs=[
            pl.BlockSpec((DIM, PC), lambda i: (0, jnp.minimum(2 * i, nbi - 1))),
            pl.BlockSpec((DIM, PC),
                         lambda i: (0, jnp.minimum(2 * i + 1, nbi - 1))),
            pl.BlockSpec((DIM, DIM), lambda i: (0, 0)),
        ],
        out_specs=pl.BlockSpec((PC, 2 * DIM), lambda i: (i, 0)),
        out_shape=jax.ShapeDtypeStruct((nb * PC, 2 * DIM), jnp.float32),
        compiler_params=pltpu.CompilerParams(
            dimension_semantics=("parallel",),
        ),
    )(tabT, tabT, ident)


def _packed_row(r):
    blk = r // (2 * PC)
    rem = r % (2 * PC)
    return 2 * (blk * PC + rem % PC) + rem // PC


def _sc_gather(ids, table, nw, steps):
    n_rows = nw * steps * G
    mesh = plsc.VectorSubcoreMesh(core_axis_name="c", subcore_axis_name="s")
    nc = mesh.num_cores

    def body(id_hbm, tab_hbm, out_hbm, idx, buf, gsem):
        wid = lax.axis_index("s") * nc + lax.axis_index("c")
        pltpu.sync_copy(id_hbm.at[wid], idx)
        row0 = wid * (steps * G)

        pltpu.async_copy(tab_hbm.at[idx.at[0]], buf.at[0], gsem)

        def step(j, carry):
            slot = lax.rem(j, 2)
            nxt = lax.rem(j + 1, 2)
            pltpu.make_async_copy(tab_hbm.at[idx.at[j]], buf.at[slot],
                                  gsem).wait()

            @pl.when(j + 1 < steps)
            def _():
                pltpu.async_copy(tab_hbm.at[idx.at[j + 1]], buf.at[nxt], gsem)

            pltpu.sync_copy(buf.at[slot], out_hbm.at[pl.ds(row0 + j * G, G)])
            return carry

        lax.fori_loop(0, steps, step, 0)

    f = pl.kernel(
        body,
        out_type=jax.ShapeDtypeStruct((n_rows, DIM), jnp.float32),
        mesh=mesh,
        scratch_types=[
            pltpu.VMEM((steps, G), jnp.int32),
            pltpu.VMEM((2, G, DIM), jnp.float32),
            pltpu.SemaphoreType.DMA,
        ],
        compiler_params=pltpu.CompilerParams(use_tc_tiling_on_sc=False),
    )
    return f(ids, table)


def _ln_mlp_half_t(x, w1g, b1bt, w2, b2t):
    mu = jnp.mean(x, axis=-1, keepdims=True)
    xc = x - mu
    var = jnp.mean(xc * xc, axis=-1, keepdims=True)
    y = xc * lax.rsqrt(var + 1e-5)
    ht = lax.dot_general(w1g, y, (((0,), (1,)), ((), ())),
                         preferred_element_type=jnp.float32) + b1bt
    ht = 0.5 * ht * (1.0 + lax.erf(ht * (1.0 / math.sqrt(2.0))))
    return lax.dot_general(w2, ht, (((0,), (0,)), ((), ())),
                           preferred_element_type=jnp.float32) + b2t


def _mlp_body(es_ref, eg_ref, w1g_ref, b1bt_ref, w2_ref, b2t_ref, o_ref):
    x = es_ref[...] + eg_ref[...]
    z0t = _ln_mlp_half_t(x[:, :DIM], w1g_ref[...], b1bt_ref[...], w2_ref[...],
                         b2t_ref[...])
    z1t = _ln_mlp_half_t(x[:, DIM:], w1g_ref[...], b1bt_ref[...], w2_ref[...],
                         b2t_ref[...])
    r = x.shape[0]
    o_ref[0, :, 0:r] = z0t
    o_ref[0, :, r:2 * r] = z1t


def _tc_mlp(es, eg, gamma, beta, W1, b1, W2, b2, B, L):
    n_wide = es.shape[0] // 2
    wide_per_l = B // 2
    esw = es.reshape(n_wide, 2 * DIM)
    egw = eg.reshape(n_wide, 2 * DIM)
    w1g = gamma[:, None] * W1
    b1bt = (beta @ W1 + b1).reshape(HID, 1)
    b2t = b2.reshape(DIM, 1)
    grid = (L,)
    full = lambda shape: pl.BlockSpec(shape, lambda i: (0,) * len(shape))
    out = pl.pallas_call(
        _mlp_body,
        grid=grid,
        in_specs=[
            pl.BlockSpec((wide_per_l, 2 * DIM), lambda i: (i, 0)),
            pl.BlockSpec((wide_per_l, 2 * DIM), lambda i: (i, 0)),
            full((DIM, HID)),
            full((HID, 1)),
            full((HID, DIM)),
            full((DIM, 1)),
        ],
        out_specs=pl.BlockSpec((1, DIM, B), lambda i: (i, 0, 0)),
        out_shape=jax.ShapeDtypeStruct((L, DIM, B), jnp.float32),
        compiler_params=pltpu.CompilerParams(
            dimension_semantics=("parallel",),
        ),
    )(esw, egw, w1g, b1bt, W2, b2t)
    return jnp.transpose(out, (2, 0, 1))


def _permute_ids(ids, nw, steps, B, L):
    t = ids.T.astype(jnp.int32)
    t = t.reshape(L, 2, B // 2)
    t = jnp.transpose(t, (0, 2, 1))
    return _packed_row(t).reshape(nw, steps, G)


def kernel(station_ids, geometry_ids, station_table, geometry_table, gamma,
           beta, W1, b1, W2, b2):
    B, L = station_ids.shape
    n = B * L
    nw = 32
    steps = n // (nw * G)
    assert steps * nw * G == n

    ids_s = _permute_ids(station_ids, nw, steps, B, L)
    ids_g = _permute_ids(geometry_ids, nw, steps, B, L)
    gpack = _wide_pack(geometry_table.T)
    gview = gpack.reshape(gpack.shape[0] * 2, DIM)
    eg = _sc_gather(ids_g, gview, nw, steps)
    spack = _wide_pack(station_table.T)
    sview = spack.reshape(spack.shape[0] * 2, DIM)
    es = _sc_gather(ids_s, sview, nw, steps)
    return _tc_mlp(es, eg, gamma, beta, W1, b1, W2, b2, B, L)
